# baseline (speedup 1.0000x reference)
import functools

import jax
import jax.numpy as jnp
from jax import lax
from jax.experimental import pallas as pl
from jax.experimental.pallas import tpu as pltpu
from jax.experimental.pallas import tpu_sc as plsc

N = 10000
NPAD = 10240
D = 128
NC = 2
NS = 16
NW = NC * NS
WIN = 128
ROWS_PER_TILE = NPAD // NS
RBLK = 1024
GRID = NPAD // RBLK


@functools.cache
def _get_mesh():
    return plsc.VectorSubcoreMesh(core_axis_name="c", subcore_axis_name="s",
                                  num_cores=NC, num_subcores=NS)


def _num_windows_per_tile(E):
    nw_t = -(-E // (NW * WIN))
    if nw_t % 2 == 0:
        nw_t += 1
    return nw_t


@functools.cache
def _make_deg_kernel(nw_t):
    @functools.partial(
        pl.kernel,
        out_type=jax.ShapeDtypeStruct((NC, NPAD, 16), jnp.float32),
        mesh=_get_mesh(),
        scratch_types=[
            pltpu.VMEM((WIN, 16), jnp.float32),
            pltpu.VMEM((WIN, 16), jnp.float32),
            pltpu.VMEM((nw_t, WIN), jnp.int32),
            pltpu.VMEM_SHARED((NPAD, 16), jnp.float32),
            pltpu.SemaphoreType.DMA,
        ],
    )
    def deg_kernel(dst_hbm, deg_hbm, ones_v, buf_v, didx, acc, sem):
        cid = lax.axis_index("c")
        sid = lax.axis_index("s")
        tid = cid * NS + sid
        initval = jnp.where(cid == 0, 1.0, 0.0).astype(jnp.float32)

        pltpu.sync_copy(dst_hbm.at[tid], didx)

        @pl.loop(0, WIN)
        def _(r):
            ones_v.at[r][...] = jnp.full((16,), 1.0, jnp.float32)
            buf_v.at[r][...] = jnp.full((16,), initval, jnp.float32)

        row0 = sid * ROWS_PER_TILE

        nchunks = ROWS_PER_TILE // WIN

        @pl.loop(0, nchunks)
        def _(k):
            pltpu.async_copy(buf_v, acc.at[pl.ds(row0 + k * WIN, WIN)], sem)

        @pl.loop(0, nchunks)
        def _(k):
            pltpu.make_async_copy(buf_v, acc.at[pl.ds(row0 + k * WIN, WIN)],
                                  sem).wait()

        plsc.subcore_barrier()

        @pl.loop(0, nw_t)
        def _(w):
            pltpu.async_copy(ones_v, acc.at[didx.at[w]], sem, add=True)

        @pl.loop(0, nw_t)
        def _(w):
            pltpu.make_async_copy(ones_v, acc.at[didx.at[w]], sem).wait()

        plsc.subcore_barrier()

        @pl.loop(0, nchunks)
        def _(k):
            pltpu.sync_copy(acc.at[pl.ds(row0 + k * WIN, WIN)], buf_v)
            pltpu.sync_copy(buf_v, deg_hbm.at[cid, pl.ds(row0 + k * WIN, WIN)])

    return deg_kernel


@functools.cache
def _make_msg_kernel(nw_t):
    assert nw_t % 2 == 1

    @functools.partial(
        pl.kernel,
        out_type=jax.ShapeDtypeStruct((NC, NPAD, D), jnp.float32),
        mesh=_get_mesh(),
        scratch_types=[
            pltpu.VMEM((WIN, D), jnp.float32),
            pltpu.VMEM((WIN, D), jnp.float32),
            pltpu.VMEM((WIN,), jnp.int32),
            pltpu.VMEM((WIN,), jnp.int32),
            pltpu.VMEM((nw_t, WIN), jnp.int32),
            pltpu.VMEM_SHARED((NPAD, D), jnp.float32),
            pltpu.SemaphoreType.DMA,
            pltpu.SemaphoreType.DMA,
            pltpu.SemaphoreType.DMA,
            pltpu.SemaphoreType.DMA,
            pltpu.SemaphoreType.DMA,
            pltpu.SemaphoreType.DMA,
        ],
    )
    def msg_kernel(hs_hbm, src_hbm, dst_hbm, out_hbm,
                   rows_a, rows_b, sidx_a, sidx_b, didx, acc,
                   gsem_a, gsem_b, isem_a, isem_b, ssem_a, ssem_b):
        cid = lax.axis_index("c")
        sid = lax.axis_index("s")
        tid = cid * NS + sid
        src_t = src_hbm.at[tid]

        pltpu.sync_copy(dst_hbm.at[tid], didx)

        @pl.loop(0, WIN)
        def _(r):
            @pl.loop(0, D // 16)
            def _(c):
                rows_a.at[r, pl.ds(c * 16, 16)][...] = jnp.zeros((16,), jnp.float32)

        row0 = sid * ROWS_PER_TILE

        @pl.loop(0, ROWS_PER_TILE // WIN)
        def _(k):
            pltpu.async_copy(rows_a, acc.at[pl.ds(row0 + k * WIN, WIN)], gsem_a)

        @pl.loop(0, ROWS_PER_TILE // WIN)
        def _(k):
            pltpu.make_async_copy(rows_a, acc.at[pl.ds(row0 + k * WIN, WIN)],
                                  gsem_a).wait()

        plsc.subcore_barrier()

        def sstart(w, sbuf, sem):
            pltpu.async_copy(src_t.at[w], sbuf, sem)

        def swait(w, sbuf, sem):
            pltpu.make_async_copy(src_t.at[w], sbuf, sem).wait()

        def gstart(sbuf, rbuf, sem):
            pltpu.async_copy(hs_hbm.at[sbuf], rbuf, sem)

        def gwait(sbuf, rbuf, sem):
            pltpu.make_async_copy(hs_hbm.at[sbuf], rbuf, sem).wait()

        def sadd_go(w, rbuf, sem):
            pltpu.async_copy(rbuf, acc.at[didx.at[w]], sem, add=True)

        def sadd_drain(w, rbuf, sem):
            pltpu.make_async_copy(rbuf, acc.at[didx.at[w]], sem).wait()

        pltpu.sync_copy(src_t.at[0], sidx_a)
        gstart(sidx_a, rows_a, gsem_a)
        sstart(1, sidx_b, isem_b)
        gwait(sidx_a, rows_a, gsem_a)
        swait(1, sidx_b, isem_b)
        gstart(sidx_b, rows_b, gsem_b)
        sstart(2, sidx_a, isem_a)
        sadd_go(0, rows_a, ssem_a)
        gwait(sidx_b, rows_b, gsem_b)
        swait(2, sidx_a, isem_a)
        sadd_drain(0, rows_a, ssem_a)
        gstart(sidx_a, rows_a, gsem_a)
        sstart(3, sidx_b, isem_b)
        sadd_go(1, rows_b, ssem_b)

        @pl.loop(1, (nw_t - 1) // 2)
        def _(i):
            w = 2 * i
            gwait(sidx_a, rows_a, gsem_a)
            swait(w + 1, sidx_b, isem_b)
            sadd_drain(w - 1, rows_b, ssem_b)
            gstart(sidx_b, rows_b, gsem_b)
            sstart(w + 2, sidx_a, isem_a)
            sadd_go(w, rows_a, ssem_a)
            gwait(sidx_b, rows_b, gsem_b)
            swait(w + 2, sidx_a, isem_a)
            sadd_drain(w, rows_a, ssem_a)
            gstart(sidx_a, rows_a, gsem_a)

            @pl.when(w + 3 < nw_t)
            def _():
                sstart(w + 3, sidx_b, isem_b)

            sadd_go(w + 1, rows_b, ssem_b)

        gwait(sidx_a, rows_a, gsem_a)
        sadd_go(nw_t - 1, rows_a, ssem_a)
        sadd_drain(nw_t - 2, rows_b, ssem_b)
        sadd_drain(nw_t - 1, rows_a, ssem_a)

        plsc.subcore_barrier()

        nchunks = ROWS_PER_TILE // WIN
        bufs = (rows_a, rows_b)
        sems = (gsem_a, gsem_b)

        def acc_sl(k):
            return acc.at[pl.ds(row0 + k * WIN, WIN)]

        pltpu.async_copy(acc_sl(0), rows_a, gsem_a)
        for k in range(nchunks):
            cur, csem = bufs[k % 2], sems[k % 2]
            pltpu.make_async_copy(acc_sl(k), cur, csem).wait()
            if k + 1 < nchunks:
                pltpu.async_copy(acc_sl(k + 1), bufs[(k + 1) % 2],
                                 sems[(k + 1) % 2])
            pltpu.sync_copy(cur, out_hbm.at[cid, pl.ds(row0 + k * WIN, WIN)])

    return msg_kernel


def _dinv_from_deg(deg_ref):
    d = deg_ref[0] + deg_ref[1]
    return lax.rsqrt(d[:, 0:1])


def _matmul_body(x_ref, w_ref, o_ref):
    o_ref[...] = jnp.dot(x_ref[...], w_ref[...],
                         preferred_element_type=jnp.float32)


def _scale_body(h_ref, deg_ref, o_ref):
    o_ref[...] = h_ref[...] * _dinv_from_deg(deg_ref)


def _stage2_body(m_ref, hs_ref, deg_ref, w_ref, b_ref, o_ref):
    dinv = _dinv_from_deg(deg_ref)
    t = (m_ref[0] + m_ref[1] + hs_ref[...]) * dinv + b_ref[...]
    g = jnp.where(t > 0.0, t, jnp.exp(jnp.minimum(t, 0.0)) - 1.0)
    o_ref[...] = jnp.dot(g, w_ref[...],
                         preferred_element_type=jnp.float32) * dinv


def _stage3_body(m_ref, hs_ref, deg_ref, b_ref, o_ref):
    dinv = _dinv_from_deg(deg_ref)
    o_ref[...] = (m_ref[0] + m_ref[1] + hs_ref[...]) * dinv + b_ref[...]


_row_spec = pl.BlockSpec((RBLK, D), lambda i: (i, 0))
_deg_spec = pl.BlockSpec((NC, RBLK, 16), lambda i: (0, i, 0))
_msg_spec = pl.BlockSpec((NC, RBLK, D), lambda i: (0, i, 0))
_w_spec = pl.BlockSpec((D, D), lambda i: (0, 0))
_b_spec = pl.BlockSpec((1, D), lambda i: (0, 0))
_out_t = jax.ShapeDtypeStruct((NPAD, D), jnp.float32)

_tc_matmul = pl.pallas_call(
    _matmul_body, grid=(GRID,),
    in_specs=[_row_spec, _w_spec], out_specs=_row_spec, out_shape=_out_t)

_tc_scale = pl.pallas_call(
    _scale_body, grid=(GRID,),
    in_specs=[_row_spec, _deg_spec], out_specs=_row_spec, out_shape=_out_t)

_tc_stage2 = pl.pallas_call(
    _stage2_body, grid=(GRID,),
    in_specs=[_msg_spec, _row_spec, _deg_spec, _w_spec, _b_spec],
    out_specs=_row_spec, out_shape=_out_t)

_tc_stage3 = pl.pallas_call(
    _stage3_body, grid=(GRID,),
    in_specs=[_msg_spec, _row_spec, _deg_spec, _b_spec],
    out_specs=_row_spec, out_shape=_out_t)


def kernel(x, edge_index, W1, b1, W2, b2):
    E = edge_index.shape[1]
    nw_t = _num_windows_per_tile(E)
    epad = nw_t * NW * WIN - E

    pad_nodes = (N + (jnp.arange(epad, dtype=jnp.int32) % (NPAD - N))
                 ).astype(jnp.int32)
    src = jnp.concatenate([edge_index[0], pad_nodes]).reshape(NW, nw_t, WIN)
    dst = jnp.concatenate([edge_index[1], pad_nodes]).reshape(NW, nw_t, WIN)

    x_pad = jnp.concatenate(
        [x, jnp.zeros((NPAD - x.shape[0], D), x.dtype)], axis=0)
    b1r = b1.reshape(1, D)
    b2r = b2.reshape(1, D)

    deg_k = _make_deg_kernel(nw_t)
    msg_k = _make_msg_kernel(nw_t)

    degp = deg_k(dst)
    h1 = _tc_matmul(x_pad, W1)
    hs1 = _tc_scale(h1, degp)
    m1 = msg_k(hs1, src, dst)
    hs2 = _tc_stage2(m1, hs1, degp, W2, b1r)
    m2 = msg_k(hs2, src, dst)
    out = _tc_stage3(m2, hs2, degp, b2r)
    return out[:x.shape[0]]


# speedup vs baseline: 1.0186x; 1.0186x over previous
import functools

import jax
import jax.numpy as jnp
from jax import lax
from jax.experimental import pallas as pl
from jax.experimental.pallas import tpu as pltpu
from jax.experimental.pallas import tpu_sc as plsc

N = 10000
NPAD = 10240
D = 128
NC = 2
NS = 16
NW = NC * NS
WIN = 128
ROWS_PER_TILE = NPAD // NS
RBLK = 1024
GRID = NPAD // RBLK


@functools.cache
def _get_mesh():
    return plsc.VectorSubcoreMesh(core_axis_name="c", subcore_axis_name="s",
                                  num_cores=NC, num_subcores=NS)


def _num_windows_per_tile(E):
    nw_t = -(-E // (NW * WIN))
    if nw_t % 2 == 0:
        nw_t += 1
    return nw_t


@functools.cache
def _make_deg_kernel(nw_t):
    @functools.partial(
        pl.kernel,
        out_type=jax.ShapeDtypeStruct((NC, NPAD, 16), jnp.float32),
        mesh=_get_mesh(),
        scratch_types=[
            pltpu.VMEM((WIN, 16), jnp.float32),
            pltpu.VMEM((WIN, 16), jnp.float32),
            pltpu.VMEM((nw_t, WIN), jnp.int32),
            pltpu.VMEM_SHARED((NPAD, 16), jnp.float32),
            pltpu.SemaphoreType.DMA,
        ],
    )
    def deg_kernel(dst_hbm, deg_hbm, ones_v, buf_v, didx, acc, sem):
        cid = lax.axis_index("c")
        sid = lax.axis_index("s")
        tid = cid * NS + sid
        initval = jnp.where(cid == 0, 1.0, 0.0).astype(jnp.float32)

        pltpu.sync_copy(dst_hbm.at[tid], didx)

        @pl.loop(0, WIN)
        def _(r):
            ones_v.at[r][...] = jnp.full((16,), 1.0, jnp.float32)
            buf_v.at[r][...] = jnp.full((16,), initval, jnp.float32)

        row0 = sid * ROWS_PER_TILE

        nchunks = ROWS_PER_TILE // WIN

        @pl.loop(0, nchunks)
        def _(k):
            pltpu.async_copy(buf_v, acc.at[pl.ds(row0 + k * WIN, WIN)], sem)

        @pl.loop(0, nchunks)
        def _(k):
            pltpu.make_async_copy(buf_v, acc.at[pl.ds(row0 + k * WIN, WIN)],
                                  sem).wait()

        plsc.subcore_barrier()

        @pl.loop(0, nw_t)
        def _(w):
            pltpu.async_copy(ones_v, acc.at[didx.at[w]], sem, add=True)

        @pl.loop(0, nw_t)
        def _(w):
            pltpu.make_async_copy(ones_v, acc.at[didx.at[w]], sem).wait()

        plsc.subcore_barrier()

        @pl.loop(0, nchunks)
        def _(k):
            pltpu.sync_copy(acc.at[pl.ds(row0 + k * WIN, WIN)], buf_v)
            pltpu.sync_copy(buf_v, deg_hbm.at[cid, pl.ds(row0 + k * WIN, WIN)])

    return deg_kernel


@functools.cache
def _make_msg_kernel(nw_t):
    assert nw_t % 2 == 1

    @functools.partial(
        pl.kernel,
        out_type=jax.ShapeDtypeStruct((NC, NPAD, D), jnp.float32),
        mesh=_get_mesh(),
        scratch_types=[
            pltpu.VMEM((WIN, D), jnp.float32),
            pltpu.VMEM((WIN, D), jnp.float32),
            pltpu.VMEM((WIN,), jnp.int32),
            pltpu.VMEM((WIN,), jnp.int32),
            pltpu.VMEM((nw_t, WIN), jnp.int32),
            pltpu.VMEM_SHARED((NPAD, D), jnp.float32),
            pltpu.SemaphoreType.DMA,
            pltpu.SemaphoreType.DMA,
            pltpu.SemaphoreType.DMA,
            pltpu.SemaphoreType.DMA,
            pltpu.SemaphoreType.DMA,
            pltpu.SemaphoreType.DMA,
        ],
    )
    def msg_kernel(hs_hbm, src_hbm, dst_hbm, out_hbm,
                   rows_a, rows_b, sidx_a, sidx_b, didx, acc,
                   gsem_a, gsem_b, isem_a, isem_b, ssem_a, ssem_b):
        cid = lax.axis_index("c")
        sid = lax.axis_index("s")
        tid = cid * NS + sid
        src_t = src_hbm.at[tid]

        pltpu.sync_copy(dst_hbm.at[tid], didx)

        @pl.loop(0, WIN)
        def _(r):
            @pl.loop(0, D // 16)
            def _(c):
                rows_a.at[r, pl.ds(c * 16, 16)][...] = jnp.zeros((16,), jnp.float32)

        row0 = sid * ROWS_PER_TILE

        @pl.loop(0, ROWS_PER_TILE // WIN)
        def _(k):
            pltpu.async_copy(rows_a, acc.at[pl.ds(row0 + k * WIN, WIN)], gsem_a)

        @pl.loop(0, ROWS_PER_TILE // WIN)
        def _(k):
            pltpu.make_async_copy(rows_a, acc.at[pl.ds(row0 + k * WIN, WIN)],
                                  gsem_a).wait()

        plsc.subcore_barrier()

        def sstart(w, sbuf, sem):
            pltpu.async_copy(src_t.at[w], sbuf, sem)

        def swait(w, sbuf, sem):
            pltpu.make_async_copy(src_t.at[w], sbuf, sem).wait()

        def gstart(sbuf, rbuf, sem):
            pltpu.async_copy(hs_hbm.at[sbuf], rbuf, sem)

        def gwait(sbuf, rbuf, sem):
            pltpu.make_async_copy(hs_hbm.at[sbuf], rbuf, sem).wait()

        def sadd_go(w, rbuf, sem):
            pltpu.async_copy(rbuf, acc.at[didx.at[w]], sem, add=True)

        def sadd_drain(w, rbuf, sem):
            pltpu.make_async_copy(rbuf, acc.at[didx.at[w]], sem).wait()

        pltpu.sync_copy(src_t.at[0], sidx_a)
        gstart(sidx_a, rows_a, gsem_a)
        sstart(1, sidx_b, isem_b)
        gwait(sidx_a, rows_a, gsem_a)
        swait(1, sidx_b, isem_b)
        gstart(sidx_b, rows_b, gsem_b)
        sstart(2, sidx_a, isem_a)
        sadd_go(0, rows_a, ssem_a)
        gwait(sidx_b, rows_b, gsem_b)
        swait(2, sidx_a, isem_a)
        sadd_drain(0, rows_a, ssem_a)
        gstart(sidx_a, rows_a, gsem_a)
        sstart(3, sidx_b, isem_b)
        sadd_go(1, rows_b, ssem_b)

        @pl.loop(1, (nw_t - 1) // 2)
        def _(i):
            w = 2 * i
            gwait(sidx_a, rows_a, gsem_a)
            swait(w + 1, sidx_b, isem_b)
            sadd_drain(w - 1, rows_b, ssem_b)
            gstart(sidx_b, rows_b, gsem_b)
            sstart(w + 2, sidx_a, isem_a)
            sadd_go(w, rows_a, ssem_a)
            gwait(sidx_b, rows_b, gsem_b)
            swait(w + 2, sidx_a, isem_a)
            sadd_drain(w, rows_a, ssem_a)
            gstart(sidx_a, rows_a, gsem_a)

            @pl.when(w + 3 < nw_t)
            def _():
                sstart(w + 3, sidx_b, isem_b)

            sadd_go(w + 1, rows_b, ssem_b)

        gwait(sidx_a, rows_a, gsem_a)
        sadd_go(nw_t - 1, rows_a, ssem_a)
        sadd_drain(nw_t - 2, rows_b, ssem_b)
        sadd_drain(nw_t - 1, rows_a, ssem_a)

        plsc.subcore_barrier()

        nchunks = ROWS_PER_TILE // WIN
        bufs = (rows_a, rows_b)
        sems = (gsem_a, gsem_b)

        def acc_sl(k):
            return acc.at[pl.ds(row0 + k * WIN, WIN)]

        pltpu.async_copy(acc_sl(0), rows_a, gsem_a)
        for k in range(nchunks):
            cur, csem = bufs[k % 2], sems[k % 2]
            pltpu.make_async_copy(acc_sl(k), cur, csem).wait()
            if k + 1 < nchunks:
                pltpu.async_copy(acc_sl(k + 1), bufs[(k + 1) % 2],
                                 sems[(k + 1) % 2])
            pltpu.sync_copy(cur, out_hbm.at[cid, pl.ds(row0 + k * WIN, WIN)])

    return msg_kernel


def _dinv_from_deg(deg_ref):
    d = deg_ref[0] + deg_ref[1]
    return lax.rsqrt(d[:, 0:1])


def _mmscale_body(x_ref, w_ref, deg_ref, o_ref):
    h = jnp.dot(x_ref[...], w_ref[...], preferred_element_type=jnp.float32)
    o_ref[...] = h * _dinv_from_deg(deg_ref)


def _stage2_body(m_ref, hs_ref, deg_ref, w_ref, b_ref, o_ref):
    dinv = _dinv_from_deg(deg_ref)
    t = (m_ref[0] + m_ref[1] + hs_ref[...]) * dinv + b_ref[...]
    g = jnp.where(t > 0.0, t, jnp.exp(jnp.minimum(t, 0.0)) - 1.0)
    o_ref[...] = jnp.dot(g, w_ref[...],
                         preferred_element_type=jnp.float32) * dinv


def _stage3_body(m_ref, hs_ref, deg_ref, b_ref, o_ref):
    dinv = _dinv_from_deg(deg_ref)
    o_ref[...] = (m_ref[0] + m_ref[1] + hs_ref[...]) * dinv + b_ref[...]


_row_spec = pl.BlockSpec((RBLK, D), lambda i: (i, 0))
_deg_spec = pl.BlockSpec((NC, RBLK, 16), lambda i: (0, i, 0))
_msg_spec = pl.BlockSpec((NC, RBLK, D), lambda i: (0, i, 0))
_w_spec = pl.BlockSpec((D, D), lambda i: (0, 0))
_b_spec = pl.BlockSpec((1, D), lambda i: (0, 0))
_out_t = jax.ShapeDtypeStruct((NPAD, D), jnp.float32)

NBLK = N // GRID
_rowN_spec = pl.BlockSpec((NBLK, D), lambda i: (i, 0))
_degN_spec = pl.BlockSpec((NC, NBLK, 16), lambda i: (0, i, 0))
_msgN_spec = pl.BlockSpec((NC, NBLK, D), lambda i: (0, i, 0))

_tc_mmscale = pl.pallas_call(
    _mmscale_body, grid=(GRID,),
    in_specs=[_row_spec, _w_spec, _deg_spec],
    out_specs=_row_spec, out_shape=_out_t)

_tc_stage2 = pl.pallas_call(
    _stage2_body, grid=(GRID,),
    in_specs=[_msg_spec, _row_spec, _deg_spec, _w_spec, _b_spec],
    out_specs=_row_spec, out_shape=_out_t)

_tc_stage3 = pl.pallas_call(
    _stage3_body, grid=(GRID,),
    in_specs=[_msgN_spec, _rowN_spec, _degN_spec, _b_spec],
    out_specs=_rowN_spec,
    out_shape=jax.ShapeDtypeStruct((N, D), jnp.float32))


def kernel(x, edge_index, W1, b1, W2, b2):
    E = edge_index.shape[1]
    nw_t = _num_windows_per_tile(E)
    epad = nw_t * NW * WIN - E

    pad_nodes = (N + (jnp.arange(epad, dtype=jnp.int32) % (NPAD - N))
                 ).astype(jnp.int32)
    src = jnp.concatenate([edge_index[0], pad_nodes]).reshape(NW, nw_t, WIN)
    dst = jnp.concatenate([edge_index[1], pad_nodes]).reshape(NW, nw_t, WIN)

    x_pad = jnp.concatenate(
        [x, jnp.zeros((NPAD - x.shape[0], D), x.dtype)], axis=0)
    b1r = b1.reshape(1, D)
    b2r = b2.reshape(1, D)

    deg_k = _make_deg_kernel(nw_t)
    msg_k = _make_msg_kernel(nw_t)

    degp = deg_k(dst)
    hs1 = _tc_mmscale(x_pad, W1, degp)
    m1 = msg_k(hs1, src, dst)
    hs2 = _tc_stage2(m1, hs1, degp, W2, b1r)
    m2 = msg_k(hs2, src, dst)
    return _tc_stage3(m2, hs2, degp, b2r)


# speedup vs baseline: 1.0352x; 1.0163x over previous
import functools

import numpy as np

import jax
import jax.numpy as jnp
from jax import lax
from jax.experimental import pallas as pl
from jax.experimental.pallas import tpu as pltpu
from jax.experimental.pallas import tpu_sc as plsc

N = 10000
NPAD = 10240
D = 128
NC = 2
NS = 16
NW = NC * NS
WIN = 128
ROWS_PER_TILE = NPAD // NS
RBLK = 2000
GRID = N // RBLK


@functools.cache
def _get_mesh():
    return plsc.VectorSubcoreMesh(core_axis_name="c", subcore_axis_name="s",
                                  num_cores=NC, num_subcores=NS)


def _num_windows_per_tile(E):
    nw_t = -(-E // (NW * WIN))
    if nw_t % 2 == 0:
        nw_t += 1
    return nw_t


@functools.cache
def _make_deg_kernel(nw_t):
    @functools.partial(
        pl.kernel,
        out_type=jax.ShapeDtypeStruct((NC, NPAD, 16), jnp.float32),
        mesh=_get_mesh(),
        scratch_types=[
            pltpu.VMEM((WIN, 16), jnp.float32),
            pltpu.VMEM((WIN, 16), jnp.float32),
            pltpu.VMEM((nw_t, WIN), jnp.int32),
            pltpu.VMEM_SHARED((NPAD, 16), jnp.float32),
            pltpu.SemaphoreType.DMA,
        ],
    )
    def deg_kernel(dst_hbm, deg_hbm, ones_v, buf_v, didx, acc, sem):
        cid = lax.axis_index("c")
        sid = lax.axis_index("s")
        tid = cid * NS + sid
        initval = jnp.where(cid == 0, 1.0, 0.0).astype(jnp.float32)

        pltpu.sync_copy(dst_hbm.at[tid], didx)

        @pl.loop(0, WIN)
        def _(r):
            ones_v.at[r][...] = jnp.full((16,), 1.0, jnp.float32)
            buf_v.at[r][...] = jnp.full((16,), initval, jnp.float32)

        row0 = sid * ROWS_PER_TILE

        nchunks = ROWS_PER_TILE // WIN

        @pl.loop(0, nchunks)
        def _(k):
            pltpu.async_copy(buf_v, acc.at[pl.ds(row0 + k * WIN, WIN)], sem)

        @pl.loop(0, nchunks)
        def _(k):
            pltpu.make_async_copy(buf_v, acc.at[pl.ds(row0 + k * WIN, WIN)],
                                  sem).wait()

        plsc.subcore_barrier()

        @pl.loop(0, nw_t)
        def _(w):
            pltpu.async_copy(ones_v, acc.at[didx.at[w]], sem, add=True)

        @pl.loop(0, nw_t)
        def _(w):
            pltpu.make_async_copy(ones_v, acc.at[didx.at[w]], sem).wait()

        plsc.subcore_barrier()

        @pl.loop(0, nchunks)
        def _(k):
            pltpu.sync_copy(acc.at[pl.ds(row0 + k * WIN, WIN)], buf_v)
            pltpu.sync_copy(buf_v, deg_hbm.at[cid, pl.ds(row0 + k * WIN, WIN)])

    return deg_kernel


@functools.cache
def _make_msg_kernel(nw_t):
    assert nw_t % 2 == 1

    @functools.partial(
        pl.kernel,
        out_type=jax.ShapeDtypeStruct((NC, NPAD, D), jnp.float32),
        mesh=_get_mesh(),
        scratch_types=[
            pltpu.VMEM((WIN, D), jnp.float32),
            pltpu.VMEM((WIN, D), jnp.float32),
            pltpu.VMEM((WIN,), jnp.int32),
            pltpu.VMEM((WIN,), jnp.int32),
            pltpu.VMEM((nw_t, WIN), jnp.int32),
            pltpu.VMEM_SHARED((NPAD, D), jnp.float32),
            pltpu.SemaphoreType.DMA,
            pltpu.SemaphoreType.DMA,
            pltpu.SemaphoreType.DMA,
            pltpu.SemaphoreType.DMA,
            pltpu.SemaphoreType.DMA,
            pltpu.SemaphoreType.DMA,
        ],
    )
    def msg_kernel(hs_hbm, src_hbm, dst_hbm, out_hbm,
                   rows_a, rows_b, sidx_a, sidx_b, didx, acc,
                   gsem_a, gsem_b, isem_a, isem_b, ssem_a, ssem_b):
        cid = lax.axis_index("c")
        sid = lax.axis_index("s")
        tid = cid * NS + sid
        src_t = src_hbm.at[tid]

        pltpu.sync_copy(dst_hbm.at[tid], didx)

        @pl.loop(0, WIN)
        def _(r):
            @pl.loop(0, D // 16)
            def _(c):
                rows_a.at[r, pl.ds(c * 16, 16)][...] = jnp.zeros((16,), jnp.float32)

        row0 = sid * ROWS_PER_TILE

        @pl.loop(0, ROWS_PER_TILE // WIN)
        def _(k):
            pltpu.async_copy(rows_a, acc.at[pl.ds(row0 + k * WIN, WIN)], gsem_a)

        @pl.loop(0, ROWS_PER_TILE // WIN)
        def _(k):
            pltpu.make_async_copy(rows_a, acc.at[pl.ds(row0 + k * WIN, WIN)],
                                  gsem_a).wait()

        plsc.subcore_barrier()

        def sstart(w, sbuf, sem):
            pltpu.async_copy(src_t.at[w], sbuf, sem)

        def swait(w, sbuf, sem):
            pltpu.make_async_copy(src_t.at[w], sbuf, sem).wait()

        def gstart(sbuf, rbuf, sem):
            pltpu.async_copy(hs_hbm.at[sbuf], rbuf, sem)

        def gwait(sbuf, rbuf, sem):
            pltpu.make_async_copy(hs_hbm.at[sbuf], rbuf, sem).wait()

        def sadd_go(w, rbuf, sem):
            pltpu.async_copy(rbuf, acc.at[didx.at[w]], sem, add=True)

        def sadd_drain(w, rbuf, sem):
            pltpu.make_async_copy(rbuf, acc.at[didx.at[w]], sem).wait()

        pltpu.sync_copy(src_t.at[0], sidx_a)
        gstart(sidx_a, rows_a, gsem_a)
        sstart(1, sidx_b, isem_b)
        gwait(sidx_a, rows_a, gsem_a)
        swait(1, sidx_b, isem_b)
        gstart(sidx_b, rows_b, gsem_b)
        sstart(2, sidx_a, isem_a)
        sadd_go(0, rows_a, ssem_a)
        gwait(sidx_b, rows_b, gsem_b)
        swait(2, sidx_a, isem_a)
        sadd_drain(0, rows_a, ssem_a)
        gstart(sidx_a, rows_a, gsem_a)
        sstart(3, sidx_b, isem_b)
        sadd_go(1, rows_b, ssem_b)

        @pl.loop(1, (nw_t - 1) // 2)
        def _(i):
            w = 2 * i
            gwait(sidx_a, rows_a, gsem_a)
            swait(w + 1, sidx_b, isem_b)
            sadd_drain(w - 1, rows_b, ssem_b)
            gstart(sidx_b, rows_b, gsem_b)
            sstart(w + 2, sidx_a, isem_a)
            sadd_go(w, rows_a, ssem_a)
            gwait(sidx_b, rows_b, gsem_b)
            swait(w + 2, sidx_a, isem_a)
            sadd_drain(w, rows_a, ssem_a)
            gstart(sidx_a, rows_a, gsem_a)

            @pl.when(w + 3 < nw_t)
            def _():
                sstart(w + 3, sidx_b, isem_b)

            sadd_go(w + 1, rows_b, ssem_b)

        gwait(sidx_a, rows_a, gsem_a)
        sadd_go(nw_t - 1, rows_a, ssem_a)
        sadd_drain(nw_t - 2, rows_b, ssem_b)
        sadd_drain(nw_t - 1, rows_a, ssem_a)

        plsc.subcore_barrier()

        nchunks = ROWS_PER_TILE // WIN
        bufs = (rows_a, rows_b)
        sems = (gsem_a, gsem_b)

        def acc_sl(k):
            return acc.at[pl.ds(row0 + k * WIN, WIN)]

        pltpu.async_copy(acc_sl(0), rows_a, gsem_a)
        for k in range(nchunks):
            cur, csem = bufs[k % 2], sems[k % 2]
            pltpu.make_async_copy(acc_sl(k), cur, csem).wait()
            if k + 1 < nchunks:
                pltpu.async_copy(acc_sl(k + 1), bufs[(k + 1) % 2],
                                 sems[(k + 1) % 2])
            pltpu.sync_copy(cur, out_hbm.at[cid, pl.ds(row0 + k * WIN, WIN)])

    return msg_kernel


def _dinv_from_deg(deg_ref):
    d = deg_ref[0] + deg_ref[1]
    return lax.rsqrt(d[:, 0:1])


def _matmul_body(x_ref, w_ref, o_ref):
    o_ref[...] = jnp.dot(x_ref[...], w_ref[...],
                         preferred_element_type=jnp.float32)


def _scale_body(h_ref, deg_ref, o_ref):
    o_ref[...] = h_ref[...] * _dinv_from_deg(deg_ref)


def _stage2_body(m_ref, hs_ref, deg_ref, w_ref, b_ref, o_ref):
    dinv = _dinv_from_deg(deg_ref)
    t = (m_ref[0] + m_ref[1] + hs_ref[...]) * dinv + b_ref[...]
    g = jnp.where(t > 0.0, t, jnp.exp(jnp.minimum(t, 0.0)) - 1.0)
    o_ref[...] = jnp.dot(g, w_ref[...],
                         preferred_element_type=jnp.float32) * dinv


def _stage3_body(m_ref, hs_ref, deg_ref, b_ref, o_ref):
    dinv = _dinv_from_deg(deg_ref)
    o_ref[...] = (m_ref[0] + m_ref[1] + hs_ref[...]) * dinv + b_ref[...]


_row_spec = pl.BlockSpec((RBLK, D), lambda i: (i, 0))
_deg_spec = pl.BlockSpec((NC, RBLK, 16), lambda i: (0, i, 0))
_msg_spec = pl.BlockSpec((NC, RBLK, D), lambda i: (0, i, 0))
_w_spec = pl.BlockSpec((D, D), lambda i: (0, 0))
_b_spec = pl.BlockSpec((1, D), lambda i: (0, 0))
_out_t = jax.ShapeDtypeStruct((NPAD, D), jnp.float32)

_tc_matmul = pl.pallas_call(
    _matmul_body, grid=(GRID,),
    in_specs=[_row_spec, _w_spec], out_specs=_row_spec, out_shape=_out_t)

_tc_scale = pl.pallas_call(
    _scale_body, grid=(GRID,),
    in_specs=[_row_spec, _deg_spec], out_specs=_row_spec, out_shape=_out_t)

_tc_stage2 = pl.pallas_call(
    _stage2_body, grid=(GRID,),
    in_specs=[_msg_spec, _row_spec, _deg_spec, _w_spec, _b_spec],
    out_specs=_row_spec, out_shape=_out_t)

_tc_stage3 = pl.pallas_call(
    _stage3_body, grid=(GRID,),
    in_specs=[_msg_spec, _row_spec, _deg_spec, _b_spec],
    out_specs=_row_spec,
    out_shape=jax.ShapeDtypeStruct((N, D), jnp.float32))


def kernel(x, edge_index, W1, b1, W2, b2):
    E = edge_index.shape[1]
    nw_t = _num_windows_per_tile(E)
    epad = nw_t * NW * WIN - E

    pad_nodes = jnp.asarray(
        N + (np.arange(epad, dtype=np.int32) % (NPAD - N)), dtype=jnp.int32)
    src = jnp.concatenate([edge_index[0], pad_nodes]).reshape(NW, nw_t, WIN)
    dst = jnp.concatenate([edge_index[1], pad_nodes]).reshape(NW, nw_t, WIN)

    b1r = b1.reshape(1, D)
    b2r = b2.reshape(1, D)

    deg_k = _make_deg_kernel(nw_t)
    msg_k = _make_msg_kernel(nw_t)

    h1 = _tc_matmul(x, W1)
    degp = deg_k(dst)
    hs1 = _tc_scale(h1, degp)
    m1 = msg_k(hs1, src, dst)
    hs2 = _tc_stage2(m1, hs1, degp, W2, b1r)
    m2 = msg_k(hs2, src, dst)
    return _tc_stage3(m2, hs2, degp, b2r)
